# baseline (speedup 1.0000x reference)
import functools

import jax
import jax.numpy as jnp
import numpy as np
from jax import lax
from jax.experimental import pallas as pl
from jax.experimental.pallas import tpu as pltpu
from jax.sharding import Mesh, PartitionSpec as P

try:
    from jax.experimental.shard_map import shard_map as _shard_map
except ImportError:
    from jax import shard_map as _shard_map

EPS = 1e-5
INV_SQRT2 = 0.7071067811865476
HALO = 16
HC = 16


def _fold_bn(conv_bias, gamma, beta, mean, var):
    scale = gamma / jnp.sqrt(var + EPS)
    bias = beta + (conv_bias - mean) * scale
    return scale, bias


def _conv_rows(src, r0, hc, wc, dil, wg, C):
    taps = [src[r0 + (ky - 1) * dil:r0 + (ky - 1) * dil + hc, :, :]
            .reshape(hc * wc, C) for ky in range(3)]
    xcol = jnp.concatenate(taps, axis=-1)
    u = jnp.dot(xcol, wg, preferred_element_type=jnp.float32)
    a = jnp.pad(u[:, 0:C], ((dil, 0), (0, 0)))[:hc * wc]
    c = jnp.pad(u[:, 2 * C:3 * C], ((0, dil), (0, 0)))[dil:]
    return a + u[:, C:2 * C] + c


def _fused_kernel(xp_ref, wA_ref, bA_ref, w9_ref, bB_ref, w3_ref, b3_ref,
                  w1_ref, b1_ref, sb2_ref, o_ref, *, H, W, C, Ca):
    PH, PW = H + 2 * HALO, W + 2 * HALO
    XH, XW = PH - 16, PW - 16
    f32 = jnp.float32
    bf16 = jnp.bfloat16

    x2 = xp_ref[0].reshape(PH * PW, Ca)
    yA = jnp.maximum(jnp.dot(x2, wA_ref[...], preferred_element_type=f32)
                     + bA_ref[...], 0.0)
    y3 = yA.reshape(PH, PW, C)
    hh = lax.broadcasted_iota(jnp.int32, (PH, PW, 1), 0)
    ww = lax.broadcasted_iota(jnp.int32, (PH, PW, 1), 1)
    inner = (hh >= HALO) & (hh < HALO + H) & (ww >= HALO) & (ww < HALO + W)
    y3 = jnp.where(inner, y3, 0.0)
    y_bf = y3.astype(bf16)

    ww2 = lax.broadcasted_iota(jnp.int32, (HC, XW, 1), 1)
    ys = y_bf[:, 8:8 + XW, :]
    chunks = []
    for h0 in range(0, XH, HC):
        v = _conv_rows(ys, h0 + 8, HC, XW, 1, w9_ref[...], C)
        z = jnp.maximum(v + bB_ref[...], 0.0).reshape(HC, XW, C)
        hh2 = lax.broadcasted_iota(jnp.int32, (HC, XW, 1), 0) + h0
        good = ((hh2 >= 8) & (hh2 < 8 + H) & (ww2 >= 8) & (ww2 < 8 + W))
        chunks.append(jnp.where(good, z, 0.0).astype(bf16))
    xb = jnp.concatenate(chunks, axis=0)

    for i0 in range(0, H, HC):
        acc = jnp.zeros((HC * XW, C), f32)
        for bi, dil in enumerate((1, 3, 5)):
            v = _conv_rows(xb, i0 + 8, HC, XW, dil, w3_ref[bi], C)
            zb = jnp.maximum(v + b3_ref[bi:bi + 1, :], 0.0)
            acc = acc + jnp.dot(zb.astype(bf16), w1_ref[bi],
                                preferred_element_type=f32)
        acc = acc.reshape(HC, XW, C)[:, 8:8 + W, :].reshape(HC * W, C)
        y = jnp.maximum(acc + b1_ref[...], 0.0)
        resid = y3[HALO + i0:HALO + i0 + HC, HALO:HALO + W, :]
        y = y + resid.reshape(HC * W, C)
        y = y * sb2_ref[0:1, :] + sb2_ref[1:2, :]
        y = 0.5 * y * (1.0 + lax.erf(y * INV_SQRT2))
        o_ref[0, i0 * W:(i0 + HC) * W, :] = y


def _regroup(w, scale):
    C = w.shape[-1]
    return jnp.transpose(w * scale[None, None, None, :],
                         (0, 2, 1, 3)).reshape(3 * C, 3 * C)


def kernel(x, conv_w, conv_b, conv_bn_gamma, conv_bn_beta, conv_bn_mean,
           conv_bn_var, d3_w, d3_b, d3_bn_gamma, d3_bn_beta, d3_bn_mean,
           d3_bn_var, d1_w, d1_b, d1_bn_gamma, d1_bn_beta, d1_bn_mean,
           d1_bn_var, dd3_w, dd3_b, dd3_bn_gamma, dd3_bn_beta, dd3_bn_mean,
           dd3_bn_var, dd5_w, dd5_b, dd5_bn_gamma, dd5_bn_beta, dd5_bn_mean,
           dd5_bn_var, c1_w, c1_b, c1_bn_gamma, c1_bn_beta, c1_bn_mean,
           c1_bn_var, out_bn_gamma, out_bn_beta, out_bn_mean, out_bn_var):
    B, Cin, H, W = x.shape
    C = conv_b.shape[0]
    bf16 = jnp.bfloat16

    sA, bA = _fold_bn(conv_b, conv_bn_gamma, conv_bn_beta, conv_bn_mean,
                      conv_bn_var)
    sB, bB = _fold_bn(d3_b, d3_bn_gamma, d3_bn_beta, d3_bn_mean, d3_bn_var)
    s1d, b1d = _fold_bn(d1_b, d1_bn_gamma, d1_bn_beta, d1_bn_mean, d1_bn_var)
    s3d, b3d = _fold_bn(dd3_b, dd3_bn_gamma, dd3_bn_beta, dd3_bn_mean,
                        dd3_bn_var)
    s5d, b5d = _fold_bn(dd5_b, dd5_bn_gamma, dd5_bn_beta, dd5_bn_mean,
                        dd5_bn_var)
    s1, b1 = _fold_bn(c1_b, c1_bn_gamma, c1_bn_beta, c1_bn_mean, c1_bn_var)
    s2 = out_bn_gamma / jnp.sqrt(out_bn_var + EPS)
    b2 = out_bn_beta - out_bn_mean * s2

    wA = (conv_w * sA[None, :]).astype(bf16)
    w9 = _regroup(d3_w, sB).astype(bf16)
    w3 = jnp.stack([_regroup(d1_w, s1d), _regroup(dd3_w, s3d),
                    _regroup(dd5_w, s5d)]).astype(bf16)
    b3 = jnp.stack([b1d, b3d, b5d])
    w1 = (c1_w.reshape(3, C, C) * s1[None, None, :]).astype(bf16)
    sb2 = jnp.stack([s2, b2])

    args = (wA, bA.reshape(1, C), w9, bB.reshape(1, C), w3, b3, w1,
            b1.reshape(1, C), sb2)
    fwd = functools.partial(_forward_shard, H=H, W=W, C=C, Ca=Cin)

    devs = jax.devices()
    nd = 2 if (len(devs) >= 2 and B % 2 == 0) else 1
    if nd == 1:
        return fwd(x, *args)
    mesh = Mesh(np.array(devs[:nd]), ('b',))
    sharded = _shard_map(
        fwd, mesh=mesh,
        in_specs=(P('b'),) + (P(),) * len(args),
        out_specs=P('b'), check_rep=False)
    return sharded(x, *args)


def _forward_shard(x, wA, bA, w9, bB, w3, b3, w1, b1, sb2, *, H, W, C, Ca):
    B = x.shape[0]
    PH, PW = H + 2 * HALO, W + 2 * HALO
    xp = jnp.pad(jnp.transpose(x, (0, 2, 3, 1)),
                 ((0, 0), (HALO, HALO), (HALO, HALO), (0, 0))
                 ).astype(jnp.bfloat16)
    kern = functools.partial(_fused_kernel, H=H, W=W, C=C, Ca=Ca)
    out = pl.pallas_call(
        kern,
        out_shape=jax.ShapeDtypeStruct((B, H * W, C), jnp.float32),
        grid=(B,),
        in_specs=[
            pl.BlockSpec((1, PH, PW, Ca), lambda b: (b, 0, 0, 0)),
            pl.BlockSpec((Ca, C), lambda b: (0, 0)),
            pl.BlockSpec((1, C), lambda b: (0, 0)),
            pl.BlockSpec((3 * C, 3 * C), lambda b: (0, 0)),
            pl.BlockSpec((1, C), lambda b: (0, 0)),
            pl.BlockSpec((3, 3 * C, 3 * C), lambda b: (0, 0, 0)),
            pl.BlockSpec((3, C), lambda b: (0, 0)),
            pl.BlockSpec((3, C, C), lambda b: (0, 0, 0)),
            pl.BlockSpec((1, C), lambda b: (0, 0)),
            pl.BlockSpec((2, C), lambda b: (0, 0)),
        ],
        out_specs=pl.BlockSpec((1, H * W, C), lambda b: (b, 0, 0)),
        compiler_params=pltpu.CompilerParams(
            dimension_semantics=("parallel",),
            vmem_limit_bytes=60 * 1024 * 1024),
    )(xp, wA, bA, w9, bB, w3, b3, w1, b1, sb2)

    return jnp.transpose(out.reshape(B, H, W, C), (0, 3, 1, 2))


# speedup vs baseline: 1.4809x; 1.4809x over previous
import functools

import jax
import jax.numpy as jnp
import numpy as np
from jax import lax
from jax.experimental import pallas as pl
from jax.experimental.pallas import tpu as pltpu
from jax.sharding import Mesh, PartitionSpec as P

try:
    from jax.experimental.shard_map import shard_map as _shard_map
except ImportError:
    from jax import shard_map as _shard_map

EPS = 1e-5
INV_SQRT2 = 0.7071067811865476
HC = 16


def _fold_bn(conv_bias, gamma, beta, mean, var):
    scale = gamma / jnp.sqrt(var + EPS)
    bias = beta + (conv_bias - mean) * scale
    return scale, bias


def _conv_rows(src, r0, wc, dil, wg, C, W):
    taps = [src[r0 + (ky - 1) * dil:r0 + (ky - 1) * dil + HC, :, :]
            .reshape(HC * wc, C) for ky in range(3)]
    xcol = jnp.concatenate(taps, axis=-1)
    u = jnp.dot(xcol, wg, preferred_element_type=jnp.float32)
    u3 = u.reshape(HC, wc, 3 * C)
    v = (u3[:, 8 - dil:8 - dil + W, 0:C]
         + u3[:, 8:8 + W, C:2 * C]
         + u3[:, 8 + dil:8 + dil + W, 2 * C:3 * C])
    return v.reshape(HC * W, C)


def _fused_kernel(xp_ref, wA_ref, bA_ref, w9_ref, bB_ref, w3_ref, b3_ref,
                  w1_ref, b1_ref, sb2_ref, o_ref, *, H, W, C, Ca):
    f32 = jnp.float32
    bf16 = jnp.bfloat16
    WF = W + 16

    x2 = xp_ref[0].reshape(H * W, Ca)
    yA = jnp.maximum(jnp.dot(x2, wA_ref[...], preferred_element_type=f32)
                     + bA_ref[...], 0.0)
    y_pad = jnp.pad(yA.astype(bf16).reshape(H, W, C),
                    ((9, 9), (8, 8), (0, 0)))

    chunks = []
    for h0 in range(0, H, HC):
        v = _conv_rows(y_pad, h0 + 9, WF, 1, w9_ref[...], C, W)
        z = jnp.maximum(v + bB_ref[...], 0.0)
        chunks.append(z.astype(bf16).reshape(HC, W, C))
    xb = jnp.pad(jnp.concatenate(chunks, axis=0),
                 ((8, 8), (8, 8), (0, 0)))

    for i0 in range(0, H, HC):
        acc = jnp.zeros((HC * W, C), f32)
        for bi, dil in enumerate((1, 3, 5)):
            v = _conv_rows(xb, i0 + 8, WF, dil, w3_ref[bi], C, W)
            zb = jnp.maximum(v + b3_ref[bi:bi + 1, :], 0.0)
            acc = acc + jnp.dot(zb.astype(bf16), w1_ref[bi],
                                preferred_element_type=f32)
        y = jnp.maximum(acc + b1_ref[...], 0.0)
        y = y + yA[i0 * W:(i0 + HC) * W, :]
        y = y * sb2_ref[0:1, :] + sb2_ref[1:2, :]
        y = 0.5 * y * (1.0 + lax.erf(y * INV_SQRT2))
        o_ref[0, i0 * W:(i0 + HC) * W, :] = y


def _regroup(w, scale):
    C = w.shape[-1]
    return jnp.transpose(w * scale[None, None, None, :],
                         (0, 2, 1, 3)).reshape(3 * C, 3 * C)


def kernel(x, conv_w, conv_b, conv_bn_gamma, conv_bn_beta, conv_bn_mean,
           conv_bn_var, d3_w, d3_b, d3_bn_gamma, d3_bn_beta, d3_bn_mean,
           d3_bn_var, d1_w, d1_b, d1_bn_gamma, d1_bn_beta, d1_bn_mean,
           d1_bn_var, dd3_w, dd3_b, dd3_bn_gamma, dd3_bn_beta, dd3_bn_mean,
           dd3_bn_var, dd5_w, dd5_b, dd5_bn_gamma, dd5_bn_beta, dd5_bn_mean,
           dd5_bn_var, c1_w, c1_b, c1_bn_gamma, c1_bn_beta, c1_bn_mean,
           c1_bn_var, out_bn_gamma, out_bn_beta, out_bn_mean, out_bn_var):
    B, Cin, H, W = x.shape
    C = conv_b.shape[0]
    bf16 = jnp.bfloat16

    sA, bA = _fold_bn(conv_b, conv_bn_gamma, conv_bn_beta, conv_bn_mean,
                      conv_bn_var)
    sB, bB = _fold_bn(d3_b, d3_bn_gamma, d3_bn_beta, d3_bn_mean, d3_bn_var)
    s1d, b1d = _fold_bn(d1_b, d1_bn_gamma, d1_bn_beta, d1_bn_mean, d1_bn_var)
    s3d, b3d = _fold_bn(dd3_b, dd3_bn_gamma, dd3_bn_beta, dd3_bn_mean,
                        dd3_bn_var)
    s5d, b5d = _fold_bn(dd5_b, dd5_bn_gamma, dd5_bn_beta, dd5_bn_mean,
                        dd5_bn_var)
    s1, b1 = _fold_bn(c1_b, c1_bn_gamma, c1_bn_beta, c1_bn_mean, c1_bn_var)
    s2 = out_bn_gamma / jnp.sqrt(out_bn_var + EPS)
    b2 = out_bn_beta - out_bn_mean * s2

    wA = (conv_w * sA[None, :]).astype(bf16)
    w9 = _regroup(d3_w, sB).astype(bf16)
    w3 = jnp.stack([_regroup(d1_w, s1d), _regroup(dd3_w, s3d),
                    _regroup(dd5_w, s5d)]).astype(bf16)
    b3 = jnp.stack([b1d, b3d, b5d])
    w1 = (c1_w.reshape(3, C, C) * s1[None, None, :]).astype(bf16)
    sb2 = jnp.stack([s2, b2])

    args = (wA, bA.reshape(1, C), w9, bB.reshape(1, C), w3, b3, w1,
            b1.reshape(1, C), sb2)
    fwd = functools.partial(_forward_shard, H=H, W=W, C=C, Ca=Cin)

    devs = jax.devices()
    nd = 2 if (len(devs) >= 2 and B % 2 == 0) else 1
    if nd == 1:
        return fwd(x, *args)
    mesh = Mesh(np.array(devs[:nd]), ('b',))
    sharded = _shard_map(
        fwd, mesh=mesh,
        in_specs=(P('b'),) + (P(),) * len(args),
        out_specs=P('b'), check_rep=False)
    return sharded(x, *args)


def _forward_shard(x, wA, bA, w9, bB, w3, b3, w1, b1, sb2, *, H, W, C, Ca):
    B = x.shape[0]
    xp = jnp.transpose(x, (0, 2, 3, 1)).astype(jnp.bfloat16)
    kern = functools.partial(_fused_kernel, H=H, W=W, C=C, Ca=Ca)
    out = pl.pallas_call(
        kern,
        out_shape=jax.ShapeDtypeStruct((B, H * W, C), jnp.float32),
        grid=(B,),
        in_specs=[
            pl.BlockSpec((1, H, W, Ca), lambda b: (b, 0, 0, 0)),
            pl.BlockSpec((Ca, C), lambda b: (0, 0)),
            pl.BlockSpec((1, C), lambda b: (0, 0)),
            pl.BlockSpec((3 * C, 3 * C), lambda b: (0, 0)),
            pl.BlockSpec((1, C), lambda b: (0, 0)),
            pl.BlockSpec((3, 3 * C, 3 * C), lambda b: (0, 0, 0)),
            pl.BlockSpec((3, C), lambda b: (0, 0)),
            pl.BlockSpec((3, C, C), lambda b: (0, 0, 0)),
            pl.BlockSpec((1, C), lambda b: (0, 0)),
            pl.BlockSpec((2, C), lambda b: (0, 0)),
        ],
        out_specs=pl.BlockSpec((1, H * W, C), lambda b: (b, 0, 0)),
        compiler_params=pltpu.CompilerParams(
            dimension_semantics=("parallel",),
            vmem_limit_bytes=60 * 1024 * 1024),
    )(xp, wA, bA, w9, bB, w3, b3, w1, b1, sb2)

    return jnp.transpose(out.reshape(B, H, W, C), (0, 3, 1, 2))
